# Initial kernel scaffold; baseline (speedup 1.0000x reference)
import jax
import jax.numpy as jnp
from jax.experimental import pallas as pl


def kernel(x, edge_index, W1, b1, gamma, beta, W2, b2, W3, b3):
    raise NotImplementedError("write your pallas kernel here")


# speedup vs baseline: 10.8521x; 10.8521x over previous
import functools

import jax
import jax.numpy as jnp
from jax import lax
from jax.experimental import pallas as pl
from jax.experimental.pallas import tpu as pltpu
from jax.experimental.pallas import tpu_sc as plsc

F32 = jnp.float32
_NW = 32
_CW = 128


def _lk(v):
    return jnp.where(v >= 0, v, 0.1 * v)


def _make_deg(N, CH, NZB):
    mesh = plsc.VectorSubcoreMesh(core_axis_name="c", subcore_axis_name="s")
    NOUT = NZB * 16 * _CW
    rows_out = NOUT // 16

    @functools.partial(
        pl.kernel,
        out_type=jax.ShapeDtypeStruct((2, NOUT, 16), F32),
        mesh=mesh,
        scratch_types=[
            pltpu.VMEM((CH, _CW), jnp.int32),
            pltpu.VMEM((_CW, 16), F32),
            pltpu.VMEM_SHARED((NOUT, 16), F32),
        ],
    )
    def deg_kernel(dst_hbm, zeros_hbm, ones_hbm, out_hbm, dst_v, ones_v, deg_sh):
        c = lax.axis_index("c")
        s = lax.axis_index("s")
        wid = s * 2 + c
        pltpu.sync_copy(zeros_hbm, ones_v)
        for k in range(NZB):
            pltpu.sync_copy(ones_v, deg_sh.at[pl.ds((s * NZB + k) * _CW, _CW)])
        pltpu.sync_copy(ones_hbm, ones_v)
        pltpu.sync_copy(dst_hbm.at[wid], dst_v)
        plsc.subcore_barrier()

        def body(ch, carry):
            pltpu.sync_copy(ones_v, deg_sh.at[dst_v.at[ch]], add=True)
            return carry

        lax.fori_loop(0, CH, body, 0)
        plsc.subcore_barrier()
        pltpu.sync_copy(deg_sh.at[pl.ds(s * rows_out, rows_out)],
                        out_hbm.at[c, pl.ds(s * rows_out, rows_out)])

    return deg_kernel


def _make_acc(N, D, CH, NZB):
    mesh = plsc.VectorSubcoreMesh(core_axis_name="c", subcore_axis_name="s")
    NOUT = NZB * 16 * _CW
    rows_out = NOUT // 16

    @functools.partial(
        pl.kernel,
        out_type=jax.ShapeDtypeStruct((2, NOUT, D), F32),
        mesh=mesh,
        scratch_types=[
            pltpu.VMEM((CH, _CW), jnp.int32),
            pltpu.VMEM((CH, _CW), jnp.int32),
            pltpu.VMEM((_CW, D), F32),
            pltpu.VMEM_SHARED((NOUT, D), F32),
        ],
    )
    def acc_kernel(hs_hbm, src_hbm, dst_hbm, zeros_hbm, out_hbm,
                   src_v, dst_v, rows_v, acc_sh):
        c = lax.axis_index("c")
        s = lax.axis_index("s")
        wid = s * 2 + c
        pltpu.sync_copy(zeros_hbm, rows_v)
        for k in range(NZB):
            pltpu.sync_copy(rows_v, acc_sh.at[pl.ds((s * NZB + k) * _CW, _CW)])
        pltpu.sync_copy(src_hbm.at[wid], src_v)
        pltpu.sync_copy(dst_hbm.at[wid], dst_v)
        plsc.subcore_barrier()

        def body(ch, carry):
            pltpu.sync_copy(hs_hbm.at[src_v.at[ch]], rows_v)
            pltpu.sync_copy(rows_v, acc_sh.at[dst_v.at[ch]], add=True)
            return carry

        lax.fori_loop(0, CH, body, 0)
        plsc.subcore_barrier()
        pltpu.sync_copy(acc_sh.at[pl.ds(s * rows_out, rows_out)],
                        out_hbm.at[c, pl.ds(s * rows_out, rows_out)])

    return acc_kernel


_DOT = dict(preferred_element_type=F32, precision=lax.Precision.HIGHEST)


def _make_mm(N, D, BLK):
    nb = N // BLK

    def body(x_ref, w1_ref, deg_ref, hs_ref):
        h = lax.dot_general(x_ref[...], w1_ref[...], (((1,), (1,)), ((), ())), **_DOT)
        deg = deg_ref[0] + deg_ref[1]
        dinv = lax.rsqrt(deg[:, 0:1] + 1.0)
        hs_ref[...] = h * dinv

    return pl.pallas_call(
        body,
        grid=(nb,),
        in_specs=[
            pl.BlockSpec((BLK, D), lambda i: (i, 0)),
            pl.BlockSpec((D, D), lambda i: (0, 0)),
            pl.BlockSpec((2, BLK, 16), lambda i: (0, i, 0)),
        ],
        out_specs=pl.BlockSpec((BLK, D), lambda i: (i, 0)),
        out_shape=jax.ShapeDtypeStruct((N, D), F32),
    )


def _make_post(N, D, H, C, BLK):
    nb = N // BLK
    CP = 16

    def body(x_ref, hs_ref, accp_ref, deg_ref, b1_ref, g_ref, be_ref,
             w2_ref, b2_ref, w3_ref, b3_ref, pred_ref, s_ref, sum_ref, sq_ref):
        p = pl.program_id(0)
        b = pl.program_id(1)
        deg = deg_ref[0] + deg_ref[1]
        dinv = lax.rsqrt(deg[:, 0:1] + 1.0)
        conv = dinv * (hs_ref[...] + accp_ref[0] + accp_ref[1]) + b1_ref[...]
        hpre = _lk(_lk(conv))

        @pl.when(jnp.logical_and(p == 0, b == 0))
        def _():
            sum_ref[...] = jnp.zeros_like(sum_ref)
            sq_ref[...] = jnp.zeros_like(sq_ref)

        @pl.when(p == 0)
        def _():
            sum_ref[...] += jnp.sum(hpre, axis=0, keepdims=True)
            sq_ref[...] += jnp.sum(hpre * hpre, axis=0, keepdims=True)

        @pl.when(p == 1)
        def _():
            mean = sum_ref[...] * (1.0 / N)
            var = sq_ref[...] * (1.0 / N) - mean * mean
            hn = (hpre - mean) * lax.rsqrt(var + 1e-5) * g_ref[...] + be_ref[...]
            r = x_ref[...] + 0.01 * _lk(hn)
            h2 = _lk(lax.dot_general(r, w2_ref[...], (((1,), (1,)), ((), ())), **_DOT)
                     + b2_ref[...])
            logits = lax.dot_general(h2, w3_ref[...], (((1,), (1,)), ((), ())), **_DOT) \
                + b3_ref[...]
            col = lax.broadcasted_iota(jnp.int32, logits.shape, 1)
            valid = col < C
            m = jnp.max(jnp.where(valid, logits, -1e30), axis=1, keepdims=True)
            e = jnp.where(valid, jnp.exp(logits - m), 0.0)
            prob = e / jnp.sum(e, axis=1, keepdims=True)
            S = 1.0 - prob
            for sh in (1, 2, 4, 8):
                S = S * jnp.concatenate(
                    [jnp.ones((BLK, sh), F32), S[:, :CP - sh]], axis=1)
            pred_ref[...] = prob
            s_ref[...] = S

    return pl.pallas_call(
        body,
        grid=(2, nb),
        in_specs=[
            pl.BlockSpec((BLK, D), lambda p, b: (b, 0)),
            pl.BlockSpec((BLK, D), lambda p, b: (b, 0)),
            pl.BlockSpec((2, BLK, D), lambda p, b: (0, b, 0)),
            pl.BlockSpec((2, BLK, 16), lambda p, b: (0, b, 0)),
            pl.BlockSpec((1, D), lambda p, b: (0, 0)),
            pl.BlockSpec((1, D), lambda p, b: (0, 0)),
            pl.BlockSpec((1, D), lambda p, b: (0, 0)),
            pl.BlockSpec((H, D), lambda p, b: (0, 0)),
            pl.BlockSpec((1, H), lambda p, b: (0, 0)),
            pl.BlockSpec((CP, H), lambda p, b: (0, 0)),
            pl.BlockSpec((1, CP), lambda p, b: (0, 0)),
        ],
        out_specs=[
            pl.BlockSpec((BLK, CP), lambda p, b: (b, 0)),
            pl.BlockSpec((BLK, CP), lambda p, b: (b, 0)),
        ],
        out_shape=[
            jax.ShapeDtypeStruct((N, CP), F32),
            jax.ShapeDtypeStruct((N, CP), F32),
        ],
        scratch_shapes=[
            pltpu.VMEM((1, D), F32),
            pltpu.VMEM((1, D), F32),
        ],
    )


def kernel(x, edge_index, W1, b1, gamma, beta, W2, b2, W3, b3):
    N, D = x.shape
    E = edge_index.shape[1]
    H = W2.shape[0]
    C = W3.shape[0]
    CP = 16
    CH = -(-E // (_NW * _CW))
    CH = -(-CH // 8) * 8
    EPAD = _NW * CH * _CW
    NZB = -(-(N + 1) // (16 * _CW))

    padv = jnp.full((EPAD - E,), N, jnp.int32)
    src3 = jnp.concatenate([edge_index[0], padv]).reshape(_NW, CH, _CW)
    dst3 = jnp.concatenate([edge_index[1], padv]).reshape(_NW, CH, _CW)
    zeros16 = jnp.zeros((_CW, 16), F32)
    ones16 = jnp.ones((_CW, 16), F32)
    zerosD = jnp.zeros((_CW, D), F32)

    deg2 = _make_deg(N, CH, NZB)(dst3, zeros16, ones16)[:, :N]
    hs = _make_mm(N, D, 2000)(x, W1, deg2)
    hs_pad = jnp.concatenate([hs, jnp.zeros((8, D), F32)], axis=0)
    accp = _make_acc(N, D, CH, NZB)(hs_pad, src3, dst3, zerosD)[:, :N]

    W3p = jnp.concatenate([W3, jnp.zeros((CP - C, H), F32)], axis=0)
    b3p = jnp.concatenate([b3, jnp.zeros((CP - C,), F32)]).reshape(1, CP)
    pred16, S16 = _make_post(N, D, H, C, 2000)(
        x, hs, accp, deg2,
        b1.reshape(1, D), gamma.reshape(1, D), beta.reshape(1, D),
        W2, b2.reshape(1, H), W3p, b3p)
    return pred16[:, :C], S16[:, :C]


# speedup vs baseline: 10.8642x; 1.0011x over previous
import functools

import jax
import jax.numpy as jnp
from jax import lax
from jax.experimental import pallas as pl
from jax.experimental.pallas import tpu as pltpu
from jax.experimental.pallas import tpu_sc as plsc

F32 = jnp.float32
_NW = 32
_CW = 128


def _lk(v):
    return jnp.where(v >= 0, v, 0.1 * v)


def _make_deg(N, CH, NZB):
    mesh = plsc.VectorSubcoreMesh(core_axis_name="c", subcore_axis_name="s")
    NOUT = NZB * 16 * _CW
    rows_out = NOUT // 16
    K = 8

    @functools.partial(
        pl.kernel,
        out_type=jax.ShapeDtypeStruct((2, NOUT, 16), F32),
        mesh=mesh,
        scratch_types=[
            pltpu.VMEM((CH, _CW), jnp.int32),
            pltpu.VMEM((_CW, 16), F32),
            pltpu.VMEM_SHARED((NOUT, 16), F32),
            pltpu.SemaphoreType.DMA,
        ],
    )
    def deg_kernel(dst_hbm, zeros_hbm, ones_hbm, out_hbm, dst_v, ones_v, deg_sh, sem):
        c = lax.axis_index("c")
        s = lax.axis_index("s")
        wid = s * 2 + c
        pltpu.sync_copy(zeros_hbm, ones_v)
        for k in range(NZB):
            pltpu.sync_copy(ones_v, deg_sh.at[pl.ds((s * NZB + k) * _CW, _CW)])
        pltpu.sync_copy(ones_hbm, ones_v)
        pltpu.sync_copy(dst_hbm.at[wid], dst_v)
        plsc.subcore_barrier()

        def body(i, carry):
            descs = [pltpu.async_copy(ones_v, deg_sh.at[dst_v.at[i * K + j]],
                                      sem, add=True) for j in range(K)]
            for d in descs:
                d.wait()
            return carry

        lax.fori_loop(0, CH // K, body, 0)
        plsc.subcore_barrier()
        pltpu.sync_copy(deg_sh.at[pl.ds(s * rows_out, rows_out)],
                        out_hbm.at[c, pl.ds(s * rows_out, rows_out)])

    return deg_kernel


def _make_acc(N, D, CH, NZB):
    mesh = plsc.VectorSubcoreMesh(core_axis_name="c", subcore_axis_name="s")
    NOUT = NZB * 16 * _CW
    rows_out = NOUT // 16

    @functools.partial(
        pl.kernel,
        out_type=jax.ShapeDtypeStruct((2, NOUT, D), F32),
        mesh=mesh,
        scratch_types=[
            pltpu.VMEM((CH, _CW), jnp.int32),
            pltpu.VMEM((CH, _CW), jnp.int32),
            pltpu.VMEM((_CW, D), F32),
            pltpu.VMEM((_CW, D), F32),
            pltpu.SemaphoreType.DMA,
            pltpu.SemaphoreType.DMA,
            pltpu.VMEM_SHARED((NOUT, D), F32),
        ],
    )
    def acc_kernel(hs_hbm, src_hbm, dst_hbm, zeros_hbm, out_hbm,
                   src_v, dst_v, rows0, rows1, gsem0, gsem1, acc_sh):
        c = lax.axis_index("c")
        s = lax.axis_index("s")
        wid = s * 2 + c
        pltpu.sync_copy(zeros_hbm, rows0)
        for k in range(NZB):
            pltpu.sync_copy(rows0, acc_sh.at[pl.ds((s * NZB + k) * _CW, _CW)])
        pltpu.sync_copy(src_hbm.at[wid], src_v)
        pltpu.sync_copy(dst_hbm.at[wid], dst_v)
        plsc.subcore_barrier()

        def body(ch, carry):
            pltpu.sync_copy(hs_hbm.at[src_v.at[ch]], rows0)
            pltpu.sync_copy(rows0, acc_sh.at[dst_v.at[ch]], add=True)
            return carry

        lax.fori_loop(0, CH, body, 0)
        plsc.subcore_barrier()
        pltpu.sync_copy(acc_sh.at[pl.ds(s * rows_out, rows_out)],
                        out_hbm.at[c, pl.ds(s * rows_out, rows_out)])

    return acc_kernel


_DOT = dict(preferred_element_type=F32, precision=lax.Precision.HIGHEST)


def _make_mm(N, D, BLK):
    nb = N // BLK

    def body(x_ref, w1_ref, deg_ref, hs_ref):
        h = lax.dot_general(x_ref[...], w1_ref[...], (((1,), (1,)), ((), ())), **_DOT)
        deg = deg_ref[0] + deg_ref[1]
        dinv = lax.rsqrt(deg[:, 0:1] + 1.0)
        hs_ref[...] = h * dinv

    return pl.pallas_call(
        body,
        grid=(nb,),
        in_specs=[
            pl.BlockSpec((BLK, D), lambda i: (i, 0)),
            pl.BlockSpec((D, D), lambda i: (0, 0)),
            pl.BlockSpec((2, BLK, 16), lambda i: (0, i, 0)),
        ],
        out_specs=pl.BlockSpec((BLK, D), lambda i: (i, 0)),
        out_shape=jax.ShapeDtypeStruct((N, D), F32),
    )


def _make_post(N, D, H, C, BLK):
    nb = N // BLK
    CP = 16

    def body(x_ref, hs_ref, accp_ref, deg_ref, b1_ref, g_ref, be_ref,
             w2_ref, b2_ref, w3_ref, b3_ref, pred_ref, s_ref, sum_ref, sq_ref):
        p = pl.program_id(0)
        b = pl.program_id(1)
        deg = deg_ref[0] + deg_ref[1]
        dinv = lax.rsqrt(deg[:, 0:1] + 1.0)
        conv = dinv * (hs_ref[...] + accp_ref[0] + accp_ref[1]) + b1_ref[...]
        hpre = _lk(_lk(conv))

        @pl.when(jnp.logical_and(p == 0, b == 0))
        def _():
            sum_ref[...] = jnp.zeros_like(sum_ref)
            sq_ref[...] = jnp.zeros_like(sq_ref)

        @pl.when(p == 0)
        def _():
            sum_ref[...] += jnp.sum(hpre, axis=0, keepdims=True)
            sq_ref[...] += jnp.sum(hpre * hpre, axis=0, keepdims=True)

        @pl.when(p == 1)
        def _():
            mean = sum_ref[...] * (1.0 / N)
            var = sq_ref[...] * (1.0 / N) - mean * mean
            hn = (hpre - mean) * lax.rsqrt(var + 1e-5) * g_ref[...] + be_ref[...]
            r = x_ref[...] + 0.01 * _lk(hn)
            h2 = _lk(lax.dot_general(r, w2_ref[...], (((1,), (1,)), ((), ())), **_DOT)
                     + b2_ref[...])
            logits = lax.dot_general(h2, w3_ref[...], (((1,), (1,)), ((), ())), **_DOT) \
                + b3_ref[...]
            col = lax.broadcasted_iota(jnp.int32, logits.shape, 1)
            valid = col < C
            m = jnp.max(jnp.where(valid, logits, -1e30), axis=1, keepdims=True)
            e = jnp.where(valid, jnp.exp(logits - m), 0.0)
            prob = e / jnp.sum(e, axis=1, keepdims=True)
            S = 1.0 - prob
            for sh in (1, 2, 4, 8):
                S = S * jnp.concatenate(
                    [jnp.ones((BLK, sh), F32), S[:, :CP - sh]], axis=1)
            pred_ref[...] = prob
            s_ref[...] = S

    return pl.pallas_call(
        body,
        grid=(2, nb),
        in_specs=[
            pl.BlockSpec((BLK, D), lambda p, b: (b, 0)),
            pl.BlockSpec((BLK, D), lambda p, b: (b, 0)),
            pl.BlockSpec((2, BLK, D), lambda p, b: (0, b, 0)),
            pl.BlockSpec((2, BLK, 16), lambda p, b: (0, b, 0)),
            pl.BlockSpec((1, D), lambda p, b: (0, 0)),
            pl.BlockSpec((1, D), lambda p, b: (0, 0)),
            pl.BlockSpec((1, D), lambda p, b: (0, 0)),
            pl.BlockSpec((H, D), lambda p, b: (0, 0)),
            pl.BlockSpec((1, H), lambda p, b: (0, 0)),
            pl.BlockSpec((CP, H), lambda p, b: (0, 0)),
            pl.BlockSpec((1, CP), lambda p, b: (0, 0)),
        ],
        out_specs=[
            pl.BlockSpec((BLK, CP), lambda p, b: (b, 0)),
            pl.BlockSpec((BLK, CP), lambda p, b: (b, 0)),
        ],
        out_shape=[
            jax.ShapeDtypeStruct((N, CP), F32),
            jax.ShapeDtypeStruct((N, CP), F32),
        ],
        scratch_shapes=[
            pltpu.VMEM((1, D), F32),
            pltpu.VMEM((1, D), F32),
        ],
    )


def kernel(x, edge_index, W1, b1, gamma, beta, W2, b2, W3, b3):
    N, D = x.shape
    E = edge_index.shape[1]
    H = W2.shape[0]
    C = W3.shape[0]
    CP = 16
    CH = -(-E // (_NW * _CW))
    CH = -(-CH // 8) * 8
    EPAD = _NW * CH * _CW
    NZB = -(-(N + 1) // (16 * _CW))

    padv = jnp.full((EPAD - E,), N, jnp.int32)
    src3 = jnp.concatenate([edge_index[0], padv]).reshape(_NW, CH, _CW)
    dst3 = jnp.concatenate([edge_index[1], padv]).reshape(_NW, CH, _CW)
    zeros16 = jnp.zeros((_CW, 16), F32)
    ones16 = jnp.ones((_CW, 16), F32)
    zerosD = jnp.zeros((_CW, D), F32)

    deg2 = _make_deg(N, CH, NZB)(dst3, zeros16, ones16)[:, :N]
    hs = _make_mm(N, D, 2000)(x, W1, deg2)
    hs_pad = jnp.concatenate([hs, jnp.zeros((8, D), F32)], axis=0)
    accp = _make_acc(N, D, CH, NZB)(hs_pad, src3, dst3, zerosD)[:, :N]

    W3p = jnp.concatenate([W3, jnp.zeros((CP - C, H), F32)], axis=0)
    b3p = jnp.concatenate([b3, jnp.zeros((CP - C,), F32)]).reshape(1, CP)
    pred16, S16 = _make_post(N, D, H, C, 2000)(
        x, hs, accp, deg2,
        b1.reshape(1, D), gamma.reshape(1, D), beta.reshape(1, D),
        W2, b2.reshape(1, H), W3p, b3p)
    return pred16[:, :C], S16[:, :C]
